# baseline (speedup 1.0000x reference)
import jax
import jax.numpy as jnp
from jax.experimental import pallas as pl
from jax.experimental.pallas import tpu as pltpu

_K = 512
_N_ITERS = 5


def _step_body(x_ref, c_ref, x2_ref, c2_ref, a_ref, s_ref, n_ref):
    x = x_ref[0]
    c = c_ref[0]
    x2 = x2_ref[0].reshape(-1, 1)
    c2 = c2_ref[0]
    iota_k = jax.lax.broadcasted_iota(jnp.int32, (1, _K), 1)
    xc = jax.lax.dot_general(
        x, c, (((1,), (1,)), ((), ())),
        preferred_element_type=jnp.float32)
    d = x2 - 2.0 * xc + c2
    amin = jnp.argmin(d, axis=1).astype(jnp.int32)
    a_ref[0] = amin[None, :]
    onehot = (amin[:, None] == iota_k).astype(jnp.float32)
    s_ref[0] = jax.lax.dot_general(
        onehot, x, (((0,), (0,)), ((), ())),
        precision=jax.lax.Precision.HIGHEST,
        preferred_element_type=jnp.float32)
    ones = jnp.ones((1, x.shape[0]), jnp.float32)
    n_ref[0] = jax.lax.dot_general(
        ones, onehot, (((1,), (0,)), ((), ())),
        preferred_element_type=jnp.float32)


def _assign_body(x_ref, c_ref, x2_ref, c2_ref, a_ref):
    x = x_ref[0]
    c = c_ref[0]
    x2 = x2_ref[0].reshape(-1, 1)
    c2 = c2_ref[0]
    xc = jax.lax.dot_general(
        x, c, (((1,), (1,)), ((), ())),
        preferred_element_type=jnp.float32)
    d = x2 - 2.0 * xc + c2
    amin = jnp.argmin(d, axis=1).astype(jnp.int32)
    a_ref[0] = amin[None, :]


def _in_specs(N, D):
    return [pl.BlockSpec((1, N, D), lambda b: (b, 0, 0)),
            pl.BlockSpec((1, _K, D), lambda b: (b, 0, 0)),
            pl.BlockSpec((1, 1, N), lambda b: (b, 0, 0)),
            pl.BlockSpec((1, 1, _K), lambda b: (b, 0, 0))]


def _step(x, centers, x2, c2):
    B, N, D = x.shape
    assign, sums, counts = pl.pallas_call(
        _step_body,
        grid=(B,),
        compiler_params=pltpu.CompilerParams(
            dimension_semantics=("parallel",)),
        in_specs=_in_specs(N, D),
        out_specs=[pl.BlockSpec((1, 1, N), lambda b: (b, 0, 0)),
                   pl.BlockSpec((1, _K, D), lambda b: (b, 0, 0)),
                   pl.BlockSpec((1, 1, _K), lambda b: (b, 0, 0))],
        out_shape=[jax.ShapeDtypeStruct((B, 1, N), jnp.int32),
                   jax.ShapeDtypeStruct((B, _K, D), jnp.float32),
                   jax.ShapeDtypeStruct((B, 1, _K), jnp.float32)],
    )(x, centers, x2.reshape(B, 1, N), c2.reshape(B, 1, _K))
    return assign.reshape(B, N), sums, counts.reshape(B, _K, 1)


def _assign(x, centers, x2, c2):
    B, N, D = x.shape
    out = pl.pallas_call(
        _assign_body,
        grid=(B,),
        compiler_params=pltpu.CompilerParams(
            dimension_semantics=("parallel",)),
        in_specs=_in_specs(N, D),
        out_specs=pl.BlockSpec((1, 1, N), lambda b: (b, 0, 0)),
        out_shape=jax.ShapeDtypeStruct((B, 1, N), jnp.int32),
    )(x, centers, x2.reshape(B, 1, N), c2.reshape(B, 1, _K))
    return out.reshape(B, N)


@jax.jit
def kernel(x):
    B, N, D = x.shape
    centers = x[:, :_K, :]
    x2 = jnp.sum(x * x, axis=-1)
    for _ in range(_N_ITERS):
        c2 = jnp.sum(centers * centers, axis=-1)
        _, sums, counts = _step(x, centers, x2, c2)
        centers = jnp.where(counts > 0, sums / jnp.maximum(counts, 1.0), centers)
    c2 = jnp.sum(centers * centers, axis=-1)
    assign = _assign(x, centers, x2, c2)
    return centers, assign


# speedup vs baseline: 1.0729x; 1.0729x over previous
import jax
import jax.numpy as jnp
from jax.experimental import pallas as pl
from jax.experimental.pallas import tpu as pltpu

_K = 512
_N_ITERS = 5


def _step_body(x_ref, c_ref, x2_ref, c2_ref, a_ref, s_ref, n_ref):
    x = x_ref[0]
    c = c_ref[0]
    x2 = x2_ref[0].reshape(-1, 1)
    c2 = c2_ref[0]
    iota_k = jax.lax.broadcasted_iota(jnp.int32, (1, _K), 1)
    xc = jax.lax.dot_general(
        x, c, (((1,), (1,)), ((), ())),
        preferred_element_type=jnp.float32)
    d = x2 - 2.0 * xc + c2
    dmin = jnp.min(d, axis=1, keepdims=True)
    amin = jnp.min(jnp.where(d == dmin, iota_k, _K), axis=1)
    a_ref[0] = amin[None, :]
    onehot = (amin[:, None] == iota_k).astype(jnp.float32)
    s_ref[0] = jax.lax.dot_general(
        onehot, x, (((0,), (0,)), ((), ())),
        precision=jax.lax.Precision.HIGHEST,
        preferred_element_type=jnp.float32)
    ones = jnp.ones((1, x.shape[0]), jnp.float32)
    n_ref[0] = jax.lax.dot_general(
        ones, onehot, (((1,), (0,)), ((), ())),
        preferred_element_type=jnp.float32)


def _assign_body(x_ref, c_ref, x2_ref, c2_ref, a_ref):
    x = x_ref[0]
    c = c_ref[0]
    x2 = x2_ref[0].reshape(-1, 1)
    c2 = c2_ref[0]
    iota_k = jax.lax.broadcasted_iota(jnp.int32, (1, _K), 1)
    xc = jax.lax.dot_general(
        x, c, (((1,), (1,)), ((), ())),
        preferred_element_type=jnp.float32)
    d = x2 - 2.0 * xc + c2
    dmin = jnp.min(d, axis=1, keepdims=True)
    amin = jnp.min(jnp.where(d == dmin, iota_k, _K), axis=1)
    a_ref[0] = amin[None, :]


def _in_specs(N, D):
    return [pl.BlockSpec((1, N, D), lambda b: (b, 0, 0)),
            pl.BlockSpec((1, _K, D), lambda b: (b, 0, 0)),
            pl.BlockSpec((1, 1, N), lambda b: (b, 0, 0)),
            pl.BlockSpec((1, 1, _K), lambda b: (b, 0, 0))]


def _step(x, centers, x2, c2):
    B, N, D = x.shape
    assign, sums, counts = pl.pallas_call(
        _step_body,
        grid=(B,),
        compiler_params=pltpu.CompilerParams(
            dimension_semantics=("parallel",)),
        in_specs=_in_specs(N, D),
        out_specs=[pl.BlockSpec((1, 1, N), lambda b: (b, 0, 0)),
                   pl.BlockSpec((1, _K, D), lambda b: (b, 0, 0)),
                   pl.BlockSpec((1, 1, _K), lambda b: (b, 0, 0))],
        out_shape=[jax.ShapeDtypeStruct((B, 1, N), jnp.int32),
                   jax.ShapeDtypeStruct((B, _K, D), jnp.float32),
                   jax.ShapeDtypeStruct((B, 1, _K), jnp.float32)],
    )(x, centers, x2.reshape(B, 1, N), c2.reshape(B, 1, _K))
    return assign.reshape(B, N), sums, counts.reshape(B, _K, 1)


def _assign(x, centers, x2, c2):
    B, N, D = x.shape
    out = pl.pallas_call(
        _assign_body,
        grid=(B,),
        compiler_params=pltpu.CompilerParams(
            dimension_semantics=("parallel",)),
        in_specs=_in_specs(N, D),
        out_specs=pl.BlockSpec((1, 1, N), lambda b: (b, 0, 0)),
        out_shape=jax.ShapeDtypeStruct((B, 1, N), jnp.int32),
    )(x, centers, x2.reshape(B, 1, N), c2.reshape(B, 1, _K))
    return out.reshape(B, N)


@jax.jit
def kernel(x):
    B, N, D = x.shape
    centers = x[:, :_K, :]
    x2 = jnp.sum(x * x, axis=-1)
    for _ in range(_N_ITERS):
        c2 = jnp.sum(centers * centers, axis=-1)
        _, sums, counts = _step(x, centers, x2, c2)
        centers = jnp.where(counts > 0, sums / jnp.maximum(counts, 1.0), centers)
    c2 = jnp.sum(centers * centers, axis=-1)
    assign = _assign(x, centers, x2, c2)
    return centers, assign


# speedup vs baseline: 1.1650x; 1.0859x over previous
import jax
import jax.numpy as jnp
from jax.experimental import pallas as pl
from jax.experimental.pallas import tpu as pltpu

_K = 512
_N_ITERS = 5


def _step_body(x_ref, c_ref, x2_ref, c2_ref, a_ref, s_ref, n_ref):
    x = x_ref[0]
    c = c_ref[0]
    x2 = x2_ref[0].reshape(-1, 1)
    c2 = c2_ref[0]
    iota_k = jax.lax.broadcasted_iota(jnp.int32, (1, _K), 1)
    xc = jax.lax.dot_general(
        x, c, (((1,), (1,)), ((), ())),
        preferred_element_type=jnp.float32)
    d = x2 - 2.0 * xc + c2
    dmin = jnp.min(d, axis=1, keepdims=True)
    amin = jnp.min(jnp.where(d == dmin, iota_k, _K), axis=1)
    a_ref[0] = amin[:, None]
    onehot = (amin[:, None] == iota_k).astype(jnp.float32)
    s_ref[0] = jax.lax.dot_general(
        onehot, x, (((0,), (0,)), ((), ())),
        precision=jax.lax.Precision.HIGHEST,
        preferred_element_type=jnp.float32)
    n_ref[0] = jnp.sum(onehot, axis=0)[None, :]


def _assign_body(x_ref, c_ref, x2_ref, c2_ref, a_ref):
    x = x_ref[0]
    c = c_ref[0]
    x2 = x2_ref[0].reshape(-1, 1)
    c2 = c2_ref[0]
    iota_k = jax.lax.broadcasted_iota(jnp.int32, (1, _K), 1)
    xc = jax.lax.dot_general(
        x, c, (((1,), (1,)), ((), ())),
        preferred_element_type=jnp.float32)
    d = x2 - 2.0 * xc + c2
    dmin = jnp.min(d, axis=1, keepdims=True)
    amin = jnp.min(jnp.where(d == dmin, iota_k, _K), axis=1)
    a_ref[0] = amin[:, None]


def _in_specs(N, D):
    return [pl.BlockSpec((1, N, D), lambda b: (b, 0, 0)),
            pl.BlockSpec((1, _K, D), lambda b: (b, 0, 0)),
            pl.BlockSpec((1, 1, N), lambda b: (b, 0, 0)),
            pl.BlockSpec((1, 1, _K), lambda b: (b, 0, 0))]


def _step(x, centers, x2, c2):
    B, N, D = x.shape
    assign, sums, counts = pl.pallas_call(
        _step_body,
        grid=(B,),
        in_specs=_in_specs(N, D),
        out_specs=[pl.BlockSpec((1, N, 1), lambda b: (b, 0, 0)),
                   pl.BlockSpec((1, _K, D), lambda b: (b, 0, 0)),
                   pl.BlockSpec((1, 1, _K), lambda b: (b, 0, 0))],
        out_shape=[jax.ShapeDtypeStruct((B, N, 1), jnp.int32),
                   jax.ShapeDtypeStruct((B, _K, D), jnp.float32),
                   jax.ShapeDtypeStruct((B, 1, _K), jnp.float32)],
    )(x, centers, x2.reshape(B, 1, N), c2.reshape(B, 1, _K))
    return assign.reshape(B, N), sums, counts.reshape(B, _K, 1)


def _assign(x, centers, x2, c2):
    B, N, D = x.shape
    out = pl.pallas_call(
        _assign_body,
        grid=(B,),
        in_specs=_in_specs(N, D),
        out_specs=pl.BlockSpec((1, N, 1), lambda b: (b, 0, 0)),
        out_shape=jax.ShapeDtypeStruct((B, N, 1), jnp.int32),
    )(x, centers, x2.reshape(B, 1, N), c2.reshape(B, 1, _K))
    return out.reshape(B, N)


@jax.jit
def kernel(x):
    B, N, D = x.shape
    centers = x[:, :_K, :]
    x2 = jnp.sum(x * x, axis=-1)
    for _ in range(_N_ITERS):
        c2 = jnp.sum(centers * centers, axis=-1)
        _, sums, counts = _step(x, centers, x2, c2)
        centers = jnp.where(counts > 0, sums / jnp.maximum(counts, 1.0), centers)
    c2 = jnp.sum(centers * centers, axis=-1)
    assign = _assign(x, centers, x2, c2)
    return centers, assign
